# baseline (speedup 1.0000x reference)
import functools
import math

import jax
import jax.numpy as jnp
from jax import lax
from jax.experimental import pallas as pl
from jax.experimental.pallas import tpu as pltpu
from jax.experimental.pallas import tpu_sc as plsc

NUM_HIDDENS = 64
SCALE = math.sqrt(NUM_HIDDENS)

_info = plsc.get_sparse_core_info()
NC, NS, L = _info.num_cores, _info.num_subcores, _info.num_lanes
NW = NC * NS

CHUNK = 1280
VB = 1024


def _transpose_table(table_t, V, D):
    grid = (V + VB - 1) // VB

    def body(a_ref, o_ref):
        a = a_ref[...]
        t = jnp.transpose(a.reshape(D, VB // 2, 2), (1, 2, 0))
        o_ref[...] = t.reshape(VB // 2, 2 * D)

    return pl.pallas_call(
        body,
        grid=(grid,),
        in_specs=[pl.BlockSpec((D, VB), lambda i: (0, i))],
        out_specs=pl.BlockSpec((VB // 2, 2 * D), lambda i: (i, 0)),
        out_shape=jax.ShapeDtypeStruct((V // 2, 2 * D), jnp.float32),
    )(table_t)


def _make_gather(B2, V2, D2):
    assert B2 % NW == 0
    b_per_w = B2 // NW
    assert b_per_w % CHUNK == 0
    nchunks = b_per_w // CHUNK
    mesh = plsc.VectorSubcoreMesh(core_axis_name="c", subcore_axis_name="s")

    @functools.partial(
        pl.kernel,
        mesh=mesh,
        out_type=jax.ShapeDtypeStruct((B2, D2), jnp.float32),
        compiler_params=pltpu.CompilerParams(use_tc_tiling_on_sc=False),
        scratch_types=[
            pltpu.VMEM((b_per_w,), jnp.int32),
            pltpu.VMEM((CHUNK, D2), jnp.float32),
            pltpu.VMEM((CHUNK, D2), jnp.float32),
            pltpu.SemaphoreType.DMA,
            pltpu.SemaphoreType.DMA,
        ],
    )
    def emb(x2_hbm, table4_hbm, out_hbm, idx_v, buf0, buf1, sem0, sem1):
        wid = lax.axis_index("s") * NC + lax.axis_index("c")
        base = wid * b_per_w
        bufs = (buf0, buf1)
        sems = (sem0, sem1)

        pltpu.sync_copy(x2_hbm.at[pl.ds(base, b_per_w)], idx_v)

        def gather(c):
            idx = idx_v.at[pl.ds(c * CHUNK, CHUNK)]
            return pltpu.async_copy(table4_hbm.at[idx], bufs[c % 2], sems[c % 2])

        UNROLL = 8

        def scale_body(buf, i, _):
            for u in range(UNROLL):
                r = i * UNROLL + u
                for j in range(D2 // L):
                    buf[r, pl.ds(j * L, L)] = buf[r, pl.ds(j * L, L)] * SCALE
            return 0

        handle = gather(0)
        for c in range(nchunks):
            nxt = gather(c + 1) if c + 1 < nchunks else None
            handle.wait()
            buf = bufs[c % 2]
            lax.fori_loop(0, CHUNK // UNROLL, functools.partial(scale_body, buf), 0)
            pltpu.sync_copy(buf, out_hbm.at[pl.ds(base + c * CHUNK, CHUNK)])
            handle = nxt

    return emb


@jax.jit
def kernel(x, table):
    B = x.shape[0] * x.shape[1]
    V, D = table.shape
    x_flat = x.reshape(B).astype(jnp.int32)
    x2 = jnp.stack([2 * x_flat, 2 * x_flat + 1], axis=-1).reshape(2 * B)
    table_rm = _transpose_table(table.T, V, D)
    table4 = table_rm.reshape(2 * V, D // 2)
    out2 = _make_gather(2 * B, 2 * V, D // 2)(x2, table4)
    return out2.reshape(x.shape[0], x.shape[1], D)


# speedup vs baseline: 8.4895x; 8.4895x over previous
import functools
import math

import jax
import jax.numpy as jnp
from jax import lax
from jax.experimental import pallas as pl
from jax.experimental.pallas import tpu as pltpu
from jax.experimental.pallas import tpu_sc as plsc

NUM_HIDDENS = 64
SCALE = math.sqrt(NUM_HIDDENS)

_info = plsc.get_sparse_core_info()
NC, NS, L = _info.num_cores, _info.num_subcores, _info.num_lanes
NW = NC * NS

CHUNK = 1280
VB = 1024


def _transpose_table(table_t, V, D):
    grid = (V + VB - 1) // VB

    def body(a_ref, o_ref):
        t = jnp.swapaxes(a_ref[...], 0, 1)
        t3 = t.reshape(VB // 2, 2, D)
        o_ref[...] = jnp.concatenate([t3[:, 0, :], t3[:, 1, :]], axis=1)

    return pl.pallas_call(
        body,
        grid=(grid,),
        in_specs=[pl.BlockSpec((D, VB), lambda i: (0, i))],
        out_specs=pl.BlockSpec((VB // 2, 2 * D), lambda i: (i, 0)),
        out_shape=jax.ShapeDtypeStruct((V // 2, 2 * D), jnp.float32),
    )(table_t)


def _make_gather(B2, V2, D2):
    assert B2 % NW == 0
    b_per_w = B2 // NW
    assert b_per_w % CHUNK == 0
    nchunks = b_per_w // CHUNK
    mesh = plsc.VectorSubcoreMesh(core_axis_name="c", subcore_axis_name="s")

    @functools.partial(
        pl.kernel,
        mesh=mesh,
        out_type=jax.ShapeDtypeStruct((B2, D2), jnp.float32),
        compiler_params=pltpu.CompilerParams(use_tc_tiling_on_sc=False),
        scratch_types=[
            pltpu.VMEM((b_per_w,), jnp.int32),
            pltpu.VMEM((CHUNK, D2), jnp.float32),
            pltpu.VMEM((CHUNK, D2), jnp.float32),
            pltpu.SemaphoreType.DMA,
            pltpu.SemaphoreType.DMA,
        ],
    )
    def emb(x2_hbm, table4_hbm, out_hbm, idx_v, buf0, buf1, sem0, sem1):
        wid = lax.axis_index("s") * NC + lax.axis_index("c")
        base = wid * b_per_w
        bufs = (buf0, buf1)
        sems = (sem0, sem1)

        pltpu.sync_copy(x2_hbm.at[pl.ds(base, b_per_w)], idx_v)

        def gather(c):
            idx = idx_v.at[pl.ds(c * CHUNK, CHUNK)]
            return pltpu.async_copy(table4_hbm.at[idx], bufs[c % 2], sems[c % 2])

        UNROLL = 8

        def scale_body(buf, i, _):
            for u in range(UNROLL):
                r = i * UNROLL + u
                for j in range(D2 // L):
                    buf[r, pl.ds(j * L, L)] = buf[r, pl.ds(j * L, L)] * SCALE
            return 0

        handle = gather(0)
        for c in range(nchunks):
            nxt = gather(c + 1) if c + 1 < nchunks else None
            handle.wait()
            buf = bufs[c % 2]
            lax.fori_loop(0, CHUNK // UNROLL, functools.partial(scale_body, buf), 0)
            pltpu.sync_copy(buf, out_hbm.at[pl.ds(base + c * CHUNK, CHUNK)])
            handle = nxt

    return emb


@jax.jit
def kernel(x, table):
    B = x.shape[0] * x.shape[1]
    V, D = table.shape
    x_flat = x.reshape(B).astype(jnp.int32)
    x2 = jnp.stack([2 * x_flat, 2 * x_flat + 1], axis=-1).reshape(2 * B)
    table_rm = _transpose_table(table.T, V, D)
    table4 = table_rm.reshape(2 * V, D // 2)
    out2 = _make_gather(2 * B, 2 * V, D // 2)(x2, table4)
    return out2.reshape(x.shape[0], x.shape[1], D)
